# baseline (speedup 1.0000x reference)
import functools

import jax
import jax.numpy as jnp
from jax import lax
from jax.experimental import pallas as pl
from jax.experimental.pallas import tpu as pltpu
from jax.experimental.pallas import tpu_sc as plsc

R1, R2 = 16, 16
N1, N2 = 256, 256
B = 16384

NC, NS, L = 2, 16, 16
NW = NC * NS
BW = B // NW
CB = BW // 128
NROW = 64
SUBQ = 16
NSUB = (NROW // SUBQ) * R1
LROWS = SUBQ * 128
LSLICE = N1 * N2 * R2 // L


def _build():
    mesh = plsc.VectorSubcoreMesh(
        core_axis_name="c", subcore_axis_name="s",
        num_cores=NC, num_subcores=NS)

    @functools.partial(
        pl.kernel,
        out_type=jax.ShapeDtypeStruct((R1 * 2, B * 8), jnp.float32),
        mesh=mesh,
        compiler_params=pltpu.CompilerParams(
            needs_layout_passes=False, use_tc_tiling_on_sc=False),
        scratch_types=[
            pltpu.VMEM((CB, 2, 128), jnp.int32),
            pltpu.VMEM((BW,), jnp.int32),
            pltpu.VMEM((BW,), jnp.int32),
            pltpu.VMEM((NROW, 128), jnp.int32),
            pltpu.VMEM((LROWS, L), jnp.float32),
            pltpu.VMEM((LROWS, L), jnp.float32),
            pltpu.VMEM((NROW * 128,), jnp.float32),
            pltpu.VMEM((NROW * 128,), jnp.float32),
            pltpu.SemaphoreType.DMA,
            pltpu.SemaphoreType.DMA,
            pltpu.SemaphoreType.DMA,
        ],
    )
    def run(idx_hbm, tab_hbm, out_hbm,
            pair_v, gbl_v, pos_v, idq, ln0, ln1, ob0, ob1,
            gsem0, gsem1, osem):
        w = lax.axis_index("s") * NC + lax.axis_index("c")
        pltpu.sync_copy(idx_hbm.at[pl.ds(w * CB, CB)], pair_v)

        iota = lax.iota(jnp.int32, L)

        for c4 in range(CB):
            for ch in range(8):
                i0 = pair_v[c4, 0, pl.ds(ch * L, L)]
                i1 = pair_v[c4, 1, pl.ds(ch * L, L)]
                o = c4 * 128 + ch * L
                gbl_v[pl.ds(o, L)] = (
                    i0 * 256 + (i1 >> 7) * 64 + ((i1 >> 4) & 7))
                pos_v[pl.ds(o, L)] = i1 & 15

        def expand(m, carry):
            c2 = (m // 32) * 128 + (m % 8) * 8
            gb0 = ((m // 8) % 4) * 128
            for ch in range(8):
                gbl = gbl_v[pl.ds(gb0 + ch * L, L)]
                idq[m, pl.ds(ch * L, L)] = gbl + c2
            return carry
        lax.fori_loop(0, NROW, expand, 0)

        lns = (ln0, ln1)
        obs = (ob0, ob1)
        gsems = (gsem0, gsem1)

        def fire(k):
            g, q = k // 4, k % 4
            tslice = tab_hbm.at[pl.ds(g * LSLICE, LSLICE)]
            lnv, sem = lns[k % 2], gsems[k % 2]
            def body(i, carry):
                pltpu.async_copy(
                    tslice.at[idq.at[q * SUBQ + i]],
                    lnv.at[pl.ds(i * 128, 128)], sem)
                return carry
            lax.fori_loop(0, SUBQ, body, 0)

        def drain_gather(k):
            pltpu.make_async_copy(
                tab_hbm.at[pl.ds(0, LROWS)], lns[k % 2], gsems[k % 2]).wait()

        def extract(k):
            g, q = k // 4, k % 4
            lnv, obv = lns[k % 2], obs[g % 2]
            def body(i, carry):
                m = q * SUBQ + i
                pbase = ((m // 8) % 4) * 128
                obase = m * 128
                lbase = i * 128
                for ch in range(8):
                    rows = lbase + ch * L + iota
                    cols = pos_v[pl.ds(pbase + ch * L, L)]
                    obv[pl.ds(obase + ch * L, L)] = (
                        plsc.load_gather(lnv, [rows, cols]))
                return carry
            lax.fori_loop(0, SUBQ, body, 0)

        def fire_out(g):
            for t in range(2):
                pltpu.async_copy(
                    obs[g % 2].at[pl.ds(t * 4096, 4096)],
                    out_hbm.at[2 * g + t, pl.ds(w * 4096, 4096)], osem)

        def drain_out():
            pltpu.make_async_copy(
                out_hbm.at[0, pl.ds(0, NROW * 128)], obs[0], osem).wait()

        fire(0)
        for k in range(NSUB):
            if k % 4 == 0 and k >= 8:
                drain_out()
            if k + 1 < NSUB:
                fire(k + 1)
            drain_gather(k)
            extract(k)
            if k % 4 == 3:
                fire_out(k // 4)
        drain_out()
        drain_out()

    return run


_tt_gather = _build()


@jax.jit
def kernel(indices, TT_core):
    idx3 = indices.reshape(128, 128, 2).transpose(0, 2, 1)
    tab2 = (TT_core.reshape(R1, N1, 2, 128, 2, 8)
            .transpose(0, 1, 4, 2, 5, 3).reshape(R1 * N1 * N2, R2))
    out3 = _tt_gather(idx3, tab2)
    return (out3.reshape(R1, 2, 128, 8, 128)
            .transpose(2, 4, 0, 1, 3).reshape(B, R1, R2))
